# baseline (speedup 1.0000x reference)
import functools

import jax
import jax.numpy as jnp
from jax import lax
from jax.experimental import pallas as pl
from jax.experimental.pallas import tpu as pltpu
from jax.experimental.pallas import tpu_sc as plsc

N_NODES = 10000
N_EDGES = 320000
D_FEAT = 128
EMB = 64
N_CLASSES = 5
N_GRAPHS = 64

NC = 2
NS = 16
NW = NC * NS
GROUP = 128
GROUPS_PER_W = 80
E_PAD = NW * GROUPS_PER_W * GROUP
DUMMY = N_NODES
ROWS_PER_TILE = 640
AGG_ROWS = NS * ROWS_PER_TILE


NBUF = 8
DEPTH = 4
CHUNKS_PER_TILE = ROWS_PER_TILE // GROUP


def _sc_agg_body(y_hbm, src_hbm, dst_hbm, zeros_hbm, out_hbm,
                 src_v, dst_v, rows_v, agg_sh, *sems):
    gsems, ssems = sems[:NBUF], sems[NBUF:]
    c = lax.axis_index("c")
    s = lax.axis_index("s")
    w = (NC - 1 - c) * NS + s
    pltpu.sync_copy(zeros_hbm, rows_v.at[0])
    for k in range(CHUNKS_PER_TILE):
        pltpu.sync_copy(rows_v.at[0],
                        agg_sh.at[pl.ds(s * ROWS_PER_TILE + k * GROUP, GROUP)])
    pltpu.sync_copy(src_hbm.at[pl.ds(w * GROUPS_PER_W, GROUPS_PER_W)], src_v)
    pltpu.sync_copy(dst_hbm.at[pl.ds(w * GROUPS_PER_W, GROUPS_PER_W)], dst_v)
    plsc.subcore_barrier()

    for b in range(DEPTH):
        pltpu.async_copy(y_hbm.at[src_v.at[b]], rows_v.at[b], gsems[b])

    lag = NBUF - DEPTH

    @pl.loop(0, GROUPS_PER_W, step=NBUF)
    def _pipe(j):
        for b in range(NBUF):
            g = j + b
            pltpu.make_async_copy(y_hbm.at[src_v.at[g]], rows_v.at[b],
                                  gsems[b]).wait()
            pltpu.async_copy(rows_v.at[b], agg_sh.at[dst_v.at[g]], ssems[b],
                             add=True)
            b2 = (b + DEPTH) % NBUF

            @pl.when(g + DEPTH < GROUPS_PER_W)
            def _prefetch():
                @pl.when(g >= lag)
                def _wait_prev_scatter():
                    pltpu.make_async_copy(rows_v.at[b2],
                                          agg_sh.at[dst_v.at[g - lag]],
                                          ssems[b2]).wait()

                pltpu.async_copy(y_hbm.at[src_v.at[g + DEPTH]], rows_v.at[b2],
                                 gsems[b2])

    for g in range(GROUPS_PER_W - NBUF, GROUPS_PER_W):
        pltpu.make_async_copy(rows_v.at[g % NBUF], agg_sh.at[dst_v.at[g]],
                              ssems[g % NBUF]).wait()

    plsc.subcore_barrier()
    def _out_slice(k):
        return out_hbm.at[pl.ds(w * ROWS_PER_TILE + k * GROUP, GROUP)]

    for k in range(CHUNKS_PER_TILE):
        b = k % NBUF
        if k >= NBUF:
            pltpu.make_async_copy(rows_v.at[b], _out_slice(k - NBUF),
                                  sems[b]).wait()
        pltpu.sync_copy(
            agg_sh.at[pl.ds(s * ROWS_PER_TILE + k * GROUP, GROUP)],
            rows_v.at[b])
        pltpu.async_copy(rows_v.at[b], _out_slice(k), sems[b])
    for k in range(max(0, CHUNKS_PER_TILE - NBUF), CHUNKS_PER_TILE):
        pltpu.make_async_copy(rows_v.at[k % NBUF], _out_slice(k),
                              sems[k % NBUF]).wait()


@functools.cache
def _get_sc_agg():
    return pl.kernel(
        _sc_agg_body,
        out_type=jax.ShapeDtypeStruct((NC * AGG_ROWS, EMB), jnp.float32),
        mesh=plsc.VectorSubcoreMesh(core_axis_name="c", subcore_axis_name="s",
                                    num_cores=NC, num_subcores=NS),
        scratch_types=[
            pltpu.VMEM((GROUPS_PER_W, GROUP), jnp.int32),
            pltpu.VMEM((GROUPS_PER_W, GROUP), jnp.int32),
            pltpu.VMEM((NBUF, GROUP, EMB), jnp.float32),
            pltpu.VMEM_SHARED((AGG_ROWS, EMB), jnp.float32),
        ] + [pltpu.SemaphoreType.DMA] * (2 * NBUF),
        compiler_params=pltpu.CompilerParams(use_tc_tiling_on_sc=False),
    )


def _mm_body(x_ref, w_ref, o_ref):
    o_ref[...] = jnp.dot(x_ref[...], w_ref[...],
                         preferred_element_type=jnp.float32)


def _mid_body(p_ref, y_ref, ba_ref, wb_ref, bb_ref, wn_ref, o_ref):
    agg = p_ref[0, :N_NODES, :] + p_ref[1, :N_NODES, :]
    t = jnp.maximum(y_ref[...] + agg + ba_ref[...], 0.0)
    h = jnp.maximum(
        jnp.dot(t, wb_ref[...], preferred_element_type=jnp.float32)
        + bb_ref[...], 0.0)
    o_ref[...] = jnp.dot(h, wn_ref[...], preferred_element_type=jnp.float32)


def _final_body(p_ref, y_ref, ba_ref, wb_ref, bb_ref, batch_ref,
                wout_ref, bout_ref, o_ref):
    agg = p_ref[0, :N_NODES, :] + p_ref[1, :N_NODES, :]
    t = jnp.maximum(y_ref[...] + agg + ba_ref[...], 0.0)
    h = jnp.maximum(
        jnp.dot(t, wb_ref[...], preferred_element_type=jnp.float32)
        + bb_ref[...], 0.0)
    gids = lax.broadcasted_iota(jnp.int32, (1, N_GRAPHS), 1)
    onehot = (batch_ref[...] == gids).astype(jnp.float32)
    sums = lax.dot_general(onehot, h, (((0,), (0,)), ((), ())),
                           preferred_element_type=jnp.float32)
    ones = jnp.ones((N_NODES, 1), jnp.float32)
    counts = lax.dot_general(onehot, ones, (((0,), (0,)), ((), ())),
                             preferred_element_type=jnp.float32)
    pooled = sums / jnp.maximum(counts, 1.0)
    o_ref[...] = (jnp.dot(pooled, wout_ref[...],
                          preferred_element_type=jnp.float32) + bout_ref[...])


def kernel(x, edge_index, batch, W1a, b1a, W1b, b1b, W2a, b2a, W2b, b2b,
           Wout, bout):
    pad = E_PAD - N_EDGES
    src_p = jnp.concatenate(
        [edge_index[0], jnp.zeros((pad,), jnp.int32)]).reshape(-1, GROUP)
    pad_dst = DUMMY + (jnp.arange(pad, dtype=jnp.int32) % (AGG_ROWS - N_NODES))
    dst_p = jnp.concatenate([edge_index[1], pad_dst]).reshape(-1, GROUP)
    zeros = jnp.zeros((GROUP, EMB), jnp.float32)
    b1a2, b1b2, b2a2, b2b2 = (b.reshape(1, EMB) for b in (b1a, b1b, b2a, b2b))
    batch2 = batch.reshape(N_NODES, 1)
    wout_p = jnp.zeros((EMB, 128), jnp.float32).at[:, :N_CLASSES].set(Wout)
    bout_p = jnp.zeros((1, 128), jnp.float32).at[0, :N_CLASSES].set(bout)

    f32 = jnp.float32
    y1 = pl.pallas_call(
        _mm_body,
        out_shape=jax.ShapeDtypeStruct((N_NODES, EMB), f32))(x, W1a)
    p1 = _get_sc_agg()(y1, src_p, dst_p, zeros).reshape(NC, AGG_ROWS, EMB)
    y2 = pl.pallas_call(
        _mid_body,
        out_shape=jax.ShapeDtypeStruct((N_NODES, EMB), f32))(
            p1, y1, b1a2, W1b, b1b2, W2a)
    p2 = _get_sc_agg()(y2, src_p, dst_p, zeros).reshape(NC, AGG_ROWS, EMB)
    out_p = pl.pallas_call(
        _final_body,
        out_shape=jax.ShapeDtypeStruct((N_GRAPHS, 128), f32))(
            p2, y2, b2a2, W2b, b2b2, batch2, wout_p, bout_p)
    return out_p[:, :N_CLASSES]


# speedup vs baseline: 2.6530x; 2.6530x over previous
import functools

import jax
import jax.numpy as jnp
from jax import lax
from jax.experimental import pallas as pl
from jax.experimental.pallas import tpu as pltpu
from jax.experimental.pallas import tpu_sc as plsc

N_NODES = 10000
N_EDGES = 320000
D_FEAT = 128
EMB = 64
N_CLASSES = 5
N_GRAPHS = 64

NC = 2
NS = 16
NW = NC * NS
GROUP = 128
GROUPS_PER_W = 80
E_PAD = NW * GROUPS_PER_W * GROUP
DUMMY = N_NODES
ROWS_PER_TILE = 640
AGG_ROWS = NS * ROWS_PER_TILE


NBUF = 8
DEPTH = 4
CHUNKS_PER_TILE = ROWS_PER_TILE // GROUP


def _sc_agg_body(y_hbm, src_hbm, dst_hbm, zeros_hbm, out_hbm,
                 src_v, dst_v, rows_v, agg_sh, *sems):
    gsems, ssems = sems[:NBUF], sems[NBUF:]
    c = lax.axis_index("c")
    s = lax.axis_index("s")
    w = c * NS + s
    pltpu.sync_copy(zeros_hbm, rows_v.at[0])
    for k in range(CHUNKS_PER_TILE):
        pltpu.sync_copy(rows_v.at[0],
                        agg_sh.at[pl.ds(s * ROWS_PER_TILE + k * GROUP, GROUP)])
    pltpu.sync_copy(src_hbm.at[pl.ds(w * GROUPS_PER_W, GROUPS_PER_W)], src_v)
    pltpu.sync_copy(dst_hbm.at[pl.ds(w * GROUPS_PER_W, GROUPS_PER_W)], dst_v)
    plsc.subcore_barrier()

    for b in range(DEPTH):
        pltpu.async_copy(y_hbm.at[src_v.at[b]], rows_v.at[b], gsems[b])

    lag = NBUF - DEPTH

    @pl.loop(0, GROUPS_PER_W, step=NBUF)
    def _pipe(j):
        for b in range(NBUF):
            g = j + b
            pltpu.make_async_copy(y_hbm.at[src_v.at[g]], rows_v.at[b],
                                  gsems[b]).wait()
            pltpu.async_copy(rows_v.at[b], agg_sh.at[dst_v.at[g]], ssems[b],
                             add=True)
            b2 = (b + DEPTH) % NBUF

            @pl.when(g + DEPTH < GROUPS_PER_W)
            def _prefetch():
                @pl.when(g >= lag)
                def _wait_prev_scatter():
                    pltpu.make_async_copy(rows_v.at[b2],
                                          agg_sh.at[dst_v.at[g - lag]],
                                          ssems[b2]).wait()

                pltpu.async_copy(y_hbm.at[src_v.at[g + DEPTH]], rows_v.at[b2],
                                 gsems[b2])

    for g in range(GROUPS_PER_W - NBUF, GROUPS_PER_W):
        pltpu.make_async_copy(rows_v.at[g % NBUF], agg_sh.at[dst_v.at[g]],
                              ssems[g % NBUF]).wait()

    plsc.subcore_barrier()
    def _out_slice(k):
        return out_hbm.at[pl.ds(w * ROWS_PER_TILE + k * GROUP, GROUP)]

    for k in range(CHUNKS_PER_TILE):
        b = k % NBUF
        if k >= NBUF:
            pltpu.make_async_copy(rows_v.at[b], _out_slice(k - NBUF),
                                  sems[b]).wait()
        pltpu.sync_copy(
            agg_sh.at[pl.ds(s * ROWS_PER_TILE + k * GROUP, GROUP)],
            rows_v.at[b])
        pltpu.async_copy(rows_v.at[b], _out_slice(k), sems[b])
    for k in range(max(0, CHUNKS_PER_TILE - NBUF), CHUNKS_PER_TILE):
        pltpu.make_async_copy(rows_v.at[k % NBUF], _out_slice(k),
                              sems[k % NBUF]).wait()


@functools.cache
def _get_sc_agg():
    return pl.kernel(
        _sc_agg_body,
        out_type=jax.ShapeDtypeStruct((NC * AGG_ROWS, EMB), jnp.float32),
        mesh=plsc.VectorSubcoreMesh(core_axis_name="c", subcore_axis_name="s",
                                    num_cores=NC, num_subcores=NS),
        scratch_types=[
            pltpu.VMEM((GROUPS_PER_W, GROUP), jnp.int32),
            pltpu.VMEM((GROUPS_PER_W, GROUP), jnp.int32),
            pltpu.VMEM((NBUF, GROUP, EMB), jnp.float32),
            pltpu.VMEM_SHARED((AGG_ROWS, EMB), jnp.float32),
        ] + [pltpu.SemaphoreType.DMA] * (2 * NBUF),
        compiler_params=pltpu.CompilerParams(use_tc_tiling_on_sc=False),
    )


def _mm_body(x_ref, w_ref, o_ref):
    o_ref[...] = jnp.dot(x_ref[...], w_ref[...],
                         preferred_element_type=jnp.float32)


def _mid_body(p_ref, y_ref, ba_ref, wb_ref, bb_ref, wn_ref, o_ref):
    agg = p_ref[0, :N_NODES, :] + p_ref[1, :N_NODES, :]
    t = jnp.maximum(y_ref[...] + agg + ba_ref[...], 0.0)
    h = jnp.maximum(
        jnp.dot(t, wb_ref[...], preferred_element_type=jnp.float32)
        + bb_ref[...], 0.0)
    o_ref[...] = jnp.dot(h, wn_ref[...], preferred_element_type=jnp.float32)


def _final_body(p_ref, y_ref, ba_ref, wb_ref, bb_ref, batch_ref,
                wout_ref, bout_ref, o_ref):
    agg = p_ref[0, :N_NODES, :] + p_ref[1, :N_NODES, :]
    t = jnp.maximum(y_ref[...] + agg + ba_ref[...], 0.0)
    h = jnp.maximum(
        jnp.dot(t, wb_ref[...], preferred_element_type=jnp.float32)
        + bb_ref[...], 0.0)
    gids = lax.broadcasted_iota(jnp.int32, (1, N_GRAPHS), 1)
    onehot = (batch_ref[...] == gids).astype(jnp.float32)
    sums = lax.dot_general(onehot, h, (((0,), (0,)), ((), ())),
                           preferred_element_type=jnp.float32)
    ones = jnp.ones((N_NODES, 1), jnp.float32)
    counts = lax.dot_general(onehot, ones, (((0,), (0,)), ((), ())),
                             preferred_element_type=jnp.float32)
    pooled = sums / jnp.maximum(counts, 1.0)
    o_ref[...] = (jnp.dot(pooled, wout_ref[...],
                          preferred_element_type=jnp.float32) + bout_ref[...])


def kernel(x, edge_index, batch, W1a, b1a, W1b, b1b, W2a, b2a, W2b, b2b,
           Wout, bout):
    pad = E_PAD - N_EDGES
    pad_src = jnp.arange(pad, dtype=jnp.int32) % N_NODES
    src_p = jnp.concatenate([edge_index[0], pad_src]).reshape(-1, GROUP)
    pad_dst = DUMMY + (jnp.arange(pad, dtype=jnp.int32) % (AGG_ROWS - N_NODES))
    dst_p = jnp.concatenate([edge_index[1], pad_dst]).reshape(-1, GROUP)
    zeros = jnp.zeros((GROUP, EMB), jnp.float32)
    b1a2, b1b2, b2a2, b2b2 = (b.reshape(1, EMB) for b in (b1a, b1b, b2a, b2b))
    batch2 = batch.reshape(N_NODES, 1)
    wout_p = jnp.zeros((EMB, 128), jnp.float32).at[:, :N_CLASSES].set(Wout)
    bout_p = jnp.zeros((1, 128), jnp.float32).at[0, :N_CLASSES].set(bout)

    f32 = jnp.float32
    y1 = pl.pallas_call(
        _mm_body,
        out_shape=jax.ShapeDtypeStruct((N_NODES, EMB), f32))(x, W1a)
    p1 = _get_sc_agg()(y1, src_p, dst_p, zeros).reshape(NC, AGG_ROWS, EMB)
    y2 = pl.pallas_call(
        _mid_body,
        out_shape=jax.ShapeDtypeStruct((N_NODES, EMB), f32))(
            p1, y1, b1a2, W1b, b1b2, W2a)
    p2 = _get_sc_agg()(y2, src_p, dst_p, zeros).reshape(NC, AGG_ROWS, EMB)
    out_p = pl.pallas_call(
        _final_body,
        out_shape=jax.ShapeDtypeStruct((N_GRAPHS, 128), f32))(
            p2, y2, b2a2, W2b, b2b2, batch2, wout_p, bout_p)
    return out_p[:, :N_CLASSES]


# speedup vs baseline: 2.6549x; 1.0007x over previous
import functools

import jax
import jax.numpy as jnp
from jax import lax
from jax.experimental import pallas as pl
from jax.experimental.pallas import tpu as pltpu
from jax.experimental.pallas import tpu_sc as plsc

N_NODES = 10000
N_EDGES = 320000
D_FEAT = 128
EMB = 64
N_CLASSES = 5
N_GRAPHS = 64

NC = 2
NS = 16
NW = NC * NS
GROUP = 128
GROUPS_PER_W = 80
E_PAD = NW * GROUPS_PER_W * GROUP
DUMMY = N_NODES
ROWS_PER_TILE = 640
AGG_ROWS = NS * ROWS_PER_TILE


NBUF = 8
DEPTH = 4
CHUNKS_PER_TILE = ROWS_PER_TILE // GROUP


def _sc_agg_body(y_hbm, src_hbm, dst_hbm, zeros_hbm, out_hbm,
                 src_v, dst_v, rows_v, agg_sh, *sems):
    gsems, ssems = sems[:NBUF], sems[NBUF:]
    c = lax.axis_index("c")
    s = lax.axis_index("s")
    w = c * NS + s
    pltpu.sync_copy(zeros_hbm, rows_v.at[0])
    for k in range(CHUNKS_PER_TILE):
        pltpu.sync_copy(rows_v.at[0],
                        agg_sh.at[pl.ds(s * ROWS_PER_TILE + k * GROUP, GROUP)])
    pltpu.sync_copy(src_hbm.at[pl.ds(w * GROUPS_PER_W, GROUPS_PER_W)], src_v)
    pltpu.sync_copy(dst_hbm.at[pl.ds(w * GROUPS_PER_W, GROUPS_PER_W)], dst_v)
    plsc.subcore_barrier()

    for b in range(DEPTH):
        pltpu.async_copy(y_hbm.at[src_v.at[b]], rows_v.at[b], gsems[b])

    lag = NBUF - DEPTH

    @pl.loop(0, GROUPS_PER_W, step=NBUF)
    def _pipe(j):
        for b in range(NBUF):
            g = j + b
            pltpu.make_async_copy(y_hbm.at[src_v.at[g]], rows_v.at[b],
                                  gsems[b]).wait()
            pltpu.async_copy(rows_v.at[b], agg_sh.at[dst_v.at[g]], ssems[b],
                             add=True)
            b2 = (b + DEPTH) % NBUF

            @pl.when(g + DEPTH < GROUPS_PER_W)
            def _prefetch():
                @pl.when(g >= lag)
                def _wait_prev_scatter():
                    pltpu.make_async_copy(rows_v.at[b2],
                                          agg_sh.at[dst_v.at[g - lag]],
                                          ssems[b2]).wait()

                pltpu.async_copy(y_hbm.at[src_v.at[g + DEPTH]], rows_v.at[b2],
                                 gsems[b2])

    for g in range(GROUPS_PER_W - NBUF, GROUPS_PER_W):
        pltpu.make_async_copy(rows_v.at[g % NBUF], agg_sh.at[dst_v.at[g]],
                              ssems[g % NBUF]).wait()

    plsc.subcore_barrier()
    def _out_slice(k):
        return out_hbm.at[pl.ds(w * ROWS_PER_TILE + k * GROUP, GROUP)]

    for k in range(CHUNKS_PER_TILE):
        b = k % NBUF
        if k >= NBUF:
            pltpu.make_async_copy(rows_v.at[b], _out_slice(k - NBUF),
                                  sems[b]).wait()
        pltpu.sync_copy(
            agg_sh.at[pl.ds(s * ROWS_PER_TILE + k * GROUP, GROUP)],
            rows_v.at[b])
        pltpu.async_copy(rows_v.at[b], _out_slice(k), sems[b])
    for k in range(max(0, CHUNKS_PER_TILE - NBUF), CHUNKS_PER_TILE):
        pltpu.make_async_copy(rows_v.at[k % NBUF], _out_slice(k),
                              sems[k % NBUF]).wait()


@functools.cache
def _get_sc_agg():
    return pl.kernel(
        _sc_agg_body,
        out_type=jax.ShapeDtypeStruct((NC * AGG_ROWS, EMB), jnp.float32),
        mesh=plsc.VectorSubcoreMesh(core_axis_name="c", subcore_axis_name="s",
                                    num_cores=NC, num_subcores=NS),
        scratch_types=[
            pltpu.VMEM((GROUPS_PER_W, GROUP), jnp.int32),
            pltpu.VMEM((GROUPS_PER_W, GROUP), jnp.int32),
            pltpu.VMEM((NBUF, GROUP, EMB), jnp.float32),
            pltpu.VMEM_SHARED((AGG_ROWS, EMB), jnp.float32),
        ] + [pltpu.SemaphoreType.DMA] * (2 * NBUF),
        compiler_params=pltpu.CompilerParams(use_tc_tiling_on_sc=False),
    )


def _mm_body(x_ref, w_ref, o_ref):
    o_ref[...] = jnp.dot(x_ref[...], w_ref[...],
                         preferred_element_type=jnp.float32)


def _mid_body(p_ref, y_ref, ba_ref, wb_ref, bb_ref, wn_ref, o_ref):
    agg = p_ref[:N_NODES, :] + p_ref[AGG_ROWS:AGG_ROWS + N_NODES, :]
    t = jnp.maximum(y_ref[...] + agg + ba_ref[...], 0.0)
    h = jnp.maximum(
        jnp.dot(t, wb_ref[...], preferred_element_type=jnp.float32)
        + bb_ref[...], 0.0)
    o_ref[...] = jnp.dot(h, wn_ref[...], preferred_element_type=jnp.float32)


def _final_body(p_ref, y_ref, ba_ref, wb_ref, bb_ref, batch_ref,
                wout_ref, bout_ref, o_ref):
    agg = p_ref[:N_NODES, :] + p_ref[AGG_ROWS:AGG_ROWS + N_NODES, :]
    t = jnp.maximum(y_ref[...] + agg + ba_ref[...], 0.0)
    h = jnp.maximum(
        jnp.dot(t, wb_ref[...], preferred_element_type=jnp.float32)
        + bb_ref[...], 0.0)
    gids = lax.broadcasted_iota(jnp.int32, (1, N_GRAPHS), 1)
    onehot = (batch_ref[...] == gids).astype(jnp.float32)
    sums = lax.dot_general(onehot, h, (((0,), (0,)), ((), ())),
                           preferred_element_type=jnp.float32)
    ones = jnp.ones((N_NODES, 1), jnp.float32)
    counts = lax.dot_general(onehot, ones, (((0,), (0,)), ((), ())),
                             preferred_element_type=jnp.float32)
    pooled = sums / jnp.maximum(counts, 1.0)
    o_ref[...] = (jnp.dot(pooled, wout_ref[...],
                          preferred_element_type=jnp.float32) + bout_ref[...])


def kernel(x, edge_index, batch, W1a, b1a, W1b, b1b, W2a, b2a, W2b, b2b,
           Wout, bout):
    pad = E_PAD - N_EDGES
    pad_src = jnp.arange(pad, dtype=jnp.int32) % N_NODES
    src_p = jnp.concatenate([edge_index[0], pad_src]).reshape(-1, GROUP)
    pad_dst = DUMMY + (jnp.arange(pad, dtype=jnp.int32) % (AGG_ROWS - N_NODES))
    dst_p = jnp.concatenate([edge_index[1], pad_dst]).reshape(-1, GROUP)
    zeros = jnp.zeros((GROUP, EMB), jnp.float32)
    b1a2, b1b2, b2a2, b2b2 = (b.reshape(1, EMB) for b in (b1a, b1b, b2a, b2b))
    batch2 = batch.reshape(N_NODES, 1)
    wout_p = jnp.zeros((EMB, 128), jnp.float32).at[:, :N_CLASSES].set(Wout)
    bout_p = jnp.zeros((1, 128), jnp.float32).at[0, :N_CLASSES].set(bout)

    f32 = jnp.float32
    y1 = pl.pallas_call(
        _mm_body,
        out_shape=jax.ShapeDtypeStruct((N_NODES, EMB), f32))(x, W1a)
    p1 = _get_sc_agg()(y1, src_p, dst_p, zeros)
    y2 = pl.pallas_call(
        _mid_body,
        out_shape=jax.ShapeDtypeStruct((N_NODES, EMB), f32))(
            p1, y1, b1a2, W1b, b1b2, W2a)
    p2 = _get_sc_agg()(y2, src_p, dst_p, zeros)
    out_p = pl.pallas_call(
        _final_body,
        out_shape=jax.ShapeDtypeStruct((N_GRAPHS, 128), f32))(
            p2, y2, b2a2, W2b, b2b2, batch2, wout_p, bout_p)
    return out_p[:, :N_CLASSES]


# speedup vs baseline: 2.7305x; 1.0285x over previous
import functools

import jax
import jax.numpy as jnp
import numpy as np
from jax import lax
from jax.experimental import pallas as pl
from jax.experimental.pallas import tpu as pltpu
from jax.experimental.pallas import tpu_sc as plsc

N_NODES = 10000
N_EDGES = 320000
D_FEAT = 128
EMB = 64
N_CLASSES = 5
N_GRAPHS = 64

NC = 2
NS = 16
NW = NC * NS
GROUP = 128
GROUPS_PER_W = 80
E_PAD = NW * GROUPS_PER_W * GROUP
DUMMY = N_NODES
ROWS_PER_TILE = 640
AGG_ROWS = NS * ROWS_PER_TILE


NBUF = 8
DEPTH = 4
CHUNKS_PER_TILE = ROWS_PER_TILE // GROUP


def _sc_agg_body(y_hbm, src_hbm, dst_hbm, zeros_hbm, out_hbm,
                 src_v, dst_v, rows_v, agg_sh, *sems):
    gsems, ssems = sems[:NBUF], sems[NBUF:]
    c = lax.axis_index("c")
    s = lax.axis_index("s")
    w = c * NS + s
    pltpu.sync_copy(zeros_hbm, rows_v.at[0])
    for k in range(CHUNKS_PER_TILE):
        pltpu.sync_copy(rows_v.at[0],
                        agg_sh.at[pl.ds(s * ROWS_PER_TILE + k * GROUP, GROUP)])
    pltpu.sync_copy(src_hbm.at[pl.ds(w * GROUPS_PER_W, GROUPS_PER_W)], src_v)
    pltpu.sync_copy(dst_hbm.at[pl.ds(w * GROUPS_PER_W, GROUPS_PER_W)], dst_v)
    plsc.subcore_barrier()

    for b in range(DEPTH):
        pltpu.async_copy(y_hbm.at[src_v.at[b]], rows_v.at[b], gsems[b])

    lag = NBUF - DEPTH

    @pl.loop(0, GROUPS_PER_W, step=NBUF)
    def _pipe(j):
        for b in range(NBUF):
            g = j + b
            pltpu.make_async_copy(y_hbm.at[src_v.at[g]], rows_v.at[b],
                                  gsems[b]).wait()
            pltpu.async_copy(rows_v.at[b], agg_sh.at[dst_v.at[g]], ssems[b],
                             add=True)
            b2 = (b + DEPTH) % NBUF

            @pl.when(g + DEPTH < GROUPS_PER_W)
            def _prefetch():
                @pl.when(g >= lag)
                def _wait_prev_scatter():
                    pltpu.make_async_copy(rows_v.at[b2],
                                          agg_sh.at[dst_v.at[g - lag]],
                                          ssems[b2]).wait()

                pltpu.async_copy(y_hbm.at[src_v.at[g + DEPTH]], rows_v.at[b2],
                                 gsems[b2])

    for g in range(GROUPS_PER_W - NBUF, GROUPS_PER_W):
        pltpu.make_async_copy(rows_v.at[g % NBUF], agg_sh.at[dst_v.at[g]],
                              ssems[g % NBUF]).wait()

    plsc.subcore_barrier()
    def _out_slice(k):
        return out_hbm.at[pl.ds(w * ROWS_PER_TILE + k * GROUP, GROUP)]

    for k in range(CHUNKS_PER_TILE):
        b = k % NBUF
        if k >= NBUF:
            pltpu.make_async_copy(rows_v.at[b], _out_slice(k - NBUF),
                                  sems[b]).wait()
        pltpu.sync_copy(
            agg_sh.at[pl.ds(s * ROWS_PER_TILE + k * GROUP, GROUP)],
            rows_v.at[b])
        pltpu.async_copy(rows_v.at[b], _out_slice(k), sems[b])
    for k in range(max(0, CHUNKS_PER_TILE - NBUF), CHUNKS_PER_TILE):
        pltpu.make_async_copy(rows_v.at[k % NBUF], _out_slice(k),
                              sems[k % NBUF]).wait()


@functools.cache
def _get_sc_agg():
    return pl.kernel(
        _sc_agg_body,
        out_type=jax.ShapeDtypeStruct((NC * AGG_ROWS, EMB), jnp.float32),
        mesh=plsc.VectorSubcoreMesh(core_axis_name="c", subcore_axis_name="s",
                                    num_cores=NC, num_subcores=NS),
        scratch_types=[
            pltpu.VMEM((GROUPS_PER_W, GROUP), jnp.int32),
            pltpu.VMEM((GROUPS_PER_W, GROUP), jnp.int32),
            pltpu.VMEM((NBUF, GROUP, EMB), jnp.float32),
            pltpu.VMEM_SHARED((AGG_ROWS, EMB), jnp.float32),
        ] + [pltpu.SemaphoreType.DMA] * (2 * NBUF),
        compiler_params=pltpu.CompilerParams(use_tc_tiling_on_sc=False),
    )


def _mm_body(x_ref, w_ref, o_ref):
    o_ref[...] = jnp.dot(x_ref[...], w_ref[...],
                         preferred_element_type=jnp.float32)


N_GROUPS = N_EDGES // GROUP
PAD_GROUPS = E_PAD // GROUP - N_GROUPS


def _pad_idx_body(ei_ref, ps_ref, pd_ref, os_ref, od_ref):
    os_ref[:N_GROUPS] = ei_ref[0]
    os_ref[N_GROUPS:] = ps_ref[...]
    od_ref[:N_GROUPS] = ei_ref[1]
    od_ref[N_GROUPS:] = pd_ref[...]


def _mid_body(p_ref, y_ref, ba_ref, wb_ref, bb_ref, wn_ref, o_ref):
    agg = p_ref[:N_NODES, :] + p_ref[AGG_ROWS:AGG_ROWS + N_NODES, :]
    t = jnp.maximum(y_ref[...] + agg + ba_ref[...], 0.0)
    h = jnp.maximum(
        jnp.dot(t, wb_ref[...], preferred_element_type=jnp.float32)
        + bb_ref[...], 0.0)
    o_ref[...] = jnp.dot(h, wn_ref[...], preferred_element_type=jnp.float32)


def _final_body(p_ref, y_ref, ba_ref, wb_ref, bb_ref, batch_ref,
                wout_ref, bout_ref, o_ref):
    agg = p_ref[:N_NODES, :] + p_ref[AGG_ROWS:AGG_ROWS + N_NODES, :]
    t = jnp.maximum(y_ref[...] + agg + ba_ref[...], 0.0)
    h = jnp.maximum(
        jnp.dot(t, wb_ref[...], preferred_element_type=jnp.float32)
        + bb_ref[...], 0.0)
    gids = lax.broadcasted_iota(jnp.int32, (1, N_GRAPHS), 1)
    onehot = (batch_ref[...] == gids).astype(jnp.float32)
    sums = lax.dot_general(onehot, h, (((0,), (0,)), ((), ())),
                           preferred_element_type=jnp.float32)
    ones = jnp.ones((N_NODES, 1), jnp.float32)
    counts = lax.dot_general(onehot, ones, (((0,), (0,)), ((), ())),
                             preferred_element_type=jnp.float32)
    pooled = sums / jnp.maximum(counts, 1.0)
    o_ref[...] = (jnp.dot(pooled, wout_ref[...],
                          preferred_element_type=jnp.float32) + bout_ref[...])


def kernel(x, edge_index, batch, W1a, b1a, W1b, b1b, W2a, b2a, W2b, b2b,
           Wout, bout):
    pad_n = E_PAD - N_EDGES
    pad_src_c = jnp.asarray(
        (np.arange(pad_n) % N_NODES).astype(np.int32).reshape(-1, GROUP))
    pad_dst_c = jnp.asarray(
        (DUMMY + np.arange(pad_n) % (AGG_ROWS - N_NODES))
        .astype(np.int32).reshape(-1, GROUP))
    i32 = jnp.int32
    src_p, dst_p = pl.pallas_call(
        _pad_idx_body,
        out_shape=[jax.ShapeDtypeStruct((E_PAD // GROUP, GROUP), i32)] * 2)(
            edge_index.reshape(2, N_GROUPS, GROUP), pad_src_c, pad_dst_c)
    zeros = jnp.zeros((GROUP, EMB), jnp.float32)
    b1a2, b1b2, b2a2, b2b2 = (b.reshape(1, EMB) for b in (b1a, b1b, b2a, b2b))
    batch2 = batch.reshape(N_NODES, 1)
    wout_p = jnp.zeros((EMB, 128), jnp.float32).at[:, :N_CLASSES].set(Wout)
    bout_p = jnp.zeros((1, 128), jnp.float32).at[0, :N_CLASSES].set(bout)

    f32 = jnp.float32
    y1 = pl.pallas_call(
        _mm_body,
        out_shape=jax.ShapeDtypeStruct((N_NODES, EMB), f32))(x, W1a)
    p1 = _get_sc_agg()(y1, src_p, dst_p, zeros)
    y2 = pl.pallas_call(
        _mid_body,
        out_shape=jax.ShapeDtypeStruct((N_NODES, EMB), f32))(
            p1, y1, b1a2, W1b, b1b2, W2a)
    p2 = _get_sc_agg()(y2, src_p, dst_p, zeros)
    out_p = pl.pallas_call(
        _final_body,
        out_shape=jax.ShapeDtypeStruct((N_GRAPHS, 128), f32))(
            p2, y2, b2a2, W2b, b2b2, batch2, wout_p, bout_p)
    return out_p[:, :N_CLASSES]


# speedup vs baseline: 2.7898x; 1.0217x over previous
import functools

import jax
import jax.numpy as jnp
import numpy as np
from jax import lax
from jax.experimental import pallas as pl
from jax.experimental.pallas import tpu as pltpu
from jax.experimental.pallas import tpu_sc as plsc

N_NODES = 10000
N_EDGES = 320000
D_FEAT = 128
EMB = 64
N_CLASSES = 5
N_GRAPHS = 64

NC = 2
NS = 16
NW = NC * NS
GROUP = 128
GROUPS_PER_W = 80
E_PAD = NW * GROUPS_PER_W * GROUP
DUMMY = N_NODES
ROWS_PER_TILE = 640
AGG_ROWS = NS * ROWS_PER_TILE


NBUF = 8
DEPTH = 5
CHUNKS_PER_TILE = ROWS_PER_TILE // GROUP


def _sc_agg_body(y_hbm, src_hbm, dst_hbm, zeros_hbm, out_hbm,
                 src_v, dst_v, rows_v, agg_sh, *sems):
    gsems, ssems = sems[:NBUF], sems[NBUF:]
    c = lax.axis_index("c")
    s = lax.axis_index("s")
    w = c * NS + s
    pltpu.sync_copy(zeros_hbm, rows_v.at[0])
    for k in range(CHUNKS_PER_TILE):
        pltpu.sync_copy(rows_v.at[0],
                        agg_sh.at[pl.ds(s * ROWS_PER_TILE + k * GROUP, GROUP)])
    pltpu.sync_copy(src_hbm.at[pl.ds(w * GROUPS_PER_W, GROUPS_PER_W)], src_v)
    pltpu.sync_copy(dst_hbm.at[pl.ds(w * GROUPS_PER_W, GROUPS_PER_W)], dst_v)
    plsc.subcore_barrier()

    for b in range(DEPTH):
        pltpu.async_copy(y_hbm.at[src_v.at[b]], rows_v.at[b], gsems[b])

    lag = NBUF - DEPTH

    @pl.loop(0, GROUPS_PER_W, step=NBUF)
    def _pipe(j):
        for b in range(NBUF):
            g = j + b
            pltpu.make_async_copy(y_hbm.at[src_v.at[g]], rows_v.at[b],
                                  gsems[b]).wait()
            pltpu.async_copy(rows_v.at[b], agg_sh.at[dst_v.at[g]], ssems[b],
                             add=True)
            b2 = (b + DEPTH) % NBUF

            @pl.when(g + DEPTH < GROUPS_PER_W)
            def _prefetch():
                @pl.when(g >= lag)
                def _wait_prev_scatter():
                    pltpu.make_async_copy(rows_v.at[b2],
                                          agg_sh.at[dst_v.at[g - lag]],
                                          ssems[b2]).wait()

                pltpu.async_copy(y_hbm.at[src_v.at[g + DEPTH]], rows_v.at[b2],
                                 gsems[b2])

    for g in range(GROUPS_PER_W - NBUF, GROUPS_PER_W):
        pltpu.make_async_copy(rows_v.at[g % NBUF], agg_sh.at[dst_v.at[g]],
                              ssems[g % NBUF]).wait()

    plsc.subcore_barrier()
    def _out_slice(k):
        return out_hbm.at[pl.ds(w * ROWS_PER_TILE + k * GROUP, GROUP)]

    for k in range(CHUNKS_PER_TILE):
        b = k % NBUF
        if k >= NBUF:
            pltpu.make_async_copy(rows_v.at[b], _out_slice(k - NBUF),
                                  sems[b]).wait()
        pltpu.sync_copy(
            agg_sh.at[pl.ds(s * ROWS_PER_TILE + k * GROUP, GROUP)],
            rows_v.at[b])
        pltpu.async_copy(rows_v.at[b], _out_slice(k), sems[b])
    for k in range(max(0, CHUNKS_PER_TILE - NBUF), CHUNKS_PER_TILE):
        pltpu.make_async_copy(rows_v.at[k % NBUF], _out_slice(k),
                              sems[k % NBUF]).wait()


@functools.cache
def _get_sc_agg():
    return pl.kernel(
        _sc_agg_body,
        out_type=jax.ShapeDtypeStruct((NC * AGG_ROWS, EMB), jnp.float32),
        mesh=plsc.VectorSubcoreMesh(core_axis_name="c", subcore_axis_name="s",
                                    num_cores=NC, num_subcores=NS),
        scratch_types=[
            pltpu.VMEM((GROUPS_PER_W, GROUP), jnp.int32),
            pltpu.VMEM((GROUPS_PER_W, GROUP), jnp.int32),
            pltpu.VMEM((NBUF, GROUP, EMB), jnp.float32),
            pltpu.VMEM_SHARED((AGG_ROWS, EMB), jnp.float32),
        ] + [pltpu.SemaphoreType.DMA] * (2 * NBUF),
        compiler_params=pltpu.CompilerParams(use_tc_tiling_on_sc=False),
    )


def _mm_body(x_ref, w_ref, o_ref):
    o_ref[...] = jnp.dot(x_ref[...], w_ref[...],
                         preferred_element_type=jnp.float32)


N_GROUPS = N_EDGES // GROUP
PAD_GROUPS = E_PAD // GROUP - N_GROUPS


def _pad_idx_body(ei_ref, ps_ref, pd_ref, os_ref, od_ref):
    os_ref[:N_GROUPS] = ei_ref[0]
    os_ref[N_GROUPS:] = ps_ref[...]
    od_ref[:N_GROUPS] = ei_ref[1]
    od_ref[N_GROUPS:] = pd_ref[...]


def _mid_body(p_ref, y_ref, ba_ref, wb_ref, bb_ref, wn_ref, o_ref):
    agg = p_ref[:N_NODES, :] + p_ref[AGG_ROWS:AGG_ROWS + N_NODES, :]
    t = jnp.maximum(y_ref[...] + agg + ba_ref[...], 0.0)
    h = jnp.maximum(
        jnp.dot(t, wb_ref[...], preferred_element_type=jnp.float32)
        + bb_ref[...], 0.0)
    o_ref[...] = jnp.dot(h, wn_ref[...], preferred_element_type=jnp.float32)


def _final_body(p_ref, y_ref, ba_ref, wb_ref, bb_ref, batch_ref,
                wout_ref, bout_ref, o_ref):
    agg = p_ref[:N_NODES, :] + p_ref[AGG_ROWS:AGG_ROWS + N_NODES, :]
    t = jnp.maximum(y_ref[...] + agg + ba_ref[...], 0.0)
    h = jnp.maximum(
        jnp.dot(t, wb_ref[...], preferred_element_type=jnp.float32)
        + bb_ref[...], 0.0)
    gids = lax.broadcasted_iota(jnp.int32, (1, N_GRAPHS), 1)
    onehot = (batch_ref[...] == gids).astype(jnp.float32)
    sums = lax.dot_general(onehot, h, (((0,), (0,)), ((), ())),
                           preferred_element_type=jnp.float32)
    ones = jnp.ones((N_NODES, 1), jnp.float32)
    counts = lax.dot_general(onehot, ones, (((0,), (0,)), ((), ())),
                             preferred_element_type=jnp.float32)
    pooled = sums / jnp.maximum(counts, 1.0)
    o_ref[...] = (jnp.dot(pooled, wout_ref[...],
                          preferred_element_type=jnp.float32) + bout_ref[...])


def kernel(x, edge_index, batch, W1a, b1a, W1b, b1b, W2a, b2a, W2b, b2b,
           Wout, bout):
    pad_n = E_PAD - N_EDGES
    pad_src_c = jnp.asarray(
        (np.arange(pad_n) % N_NODES).astype(np.int32).reshape(-1, GROUP))
    pad_dst_c = jnp.asarray(
        (DUMMY + np.arange(pad_n) % (AGG_ROWS - N_NODES))
        .astype(np.int32).reshape(-1, GROUP))
    i32 = jnp.int32
    src_p, dst_p = pl.pallas_call(
        _pad_idx_body,
        out_shape=[jax.ShapeDtypeStruct((E_PAD // GROUP, GROUP), i32)] * 2)(
            edge_index.reshape(2, N_GROUPS, GROUP), pad_src_c, pad_dst_c)
    zeros = jnp.zeros((GROUP, EMB), jnp.float32)
    b1a2, b1b2, b2a2, b2b2 = (b.reshape(1, EMB) for b in (b1a, b1b, b2a, b2b))
    batch2 = batch.reshape(N_NODES, 1)
    wout_p = jnp.zeros((EMB, 128), jnp.float32).at[:, :N_CLASSES].set(Wout)
    bout_p = jnp.zeros((1, 128), jnp.float32).at[0, :N_CLASSES].set(bout)

    f32 = jnp.float32
    y1 = pl.pallas_call(
        _mm_body,
        out_shape=jax.ShapeDtypeStruct((N_NODES, EMB), f32))(x, W1a)
    p1 = _get_sc_agg()(y1, src_p, dst_p, zeros)
    y2 = pl.pallas_call(
        _mid_body,
        out_shape=jax.ShapeDtypeStruct((N_NODES, EMB), f32))(
            p1, y1, b1a2, W1b, b1b2, W2a)
    p2 = _get_sc_agg()(y2, src_p, dst_p, zeros)
    out_p = pl.pallas_call(
        _final_body,
        out_shape=jax.ShapeDtypeStruct((N_GRAPHS, 128), f32))(
            p2, y2, b2a2, W2b, b2b2, batch2, wout_p, bout_p)
    return out_p[:, :N_CLASSES]


# speedup vs baseline: 2.8723x; 1.0296x over previous
import functools

import jax
import jax.numpy as jnp
import numpy as np
from jax import lax
from jax.experimental import pallas as pl
from jax.experimental.pallas import tpu as pltpu
from jax.experimental.pallas import tpu_sc as plsc

N_NODES = 10000
N_EDGES = 320000
D_FEAT = 128
EMB = 64
N_CLASSES = 5
N_GRAPHS = 64

NC = 2
NS = 16
NW = NC * NS
GROUP = 128
GROUPS_PER_W = 80
E_PAD = NW * GROUPS_PER_W * GROUP
DUMMY = N_NODES
ROWS_PER_TILE = 640
AGG_ROWS = NS * ROWS_PER_TILE


NBUF = 8
DEPTH = 6
CHUNKS_PER_TILE = ROWS_PER_TILE // GROUP


def _sc_agg_body(y_hbm, src_hbm, dst_hbm, zeros_hbm, out_hbm,
                 src_v, dst_v, rows_v, agg_sh, *sems):
    gsems, ssems = sems[:NBUF], sems[NBUF:]
    c = lax.axis_index("c")
    s = lax.axis_index("s")
    w = c * NS + s
    pltpu.sync_copy(zeros_hbm, rows_v.at[0])
    for k in range(CHUNKS_PER_TILE):
        pltpu.sync_copy(rows_v.at[0],
                        agg_sh.at[pl.ds(s * ROWS_PER_TILE + k * GROUP, GROUP)])
    pltpu.sync_copy(src_hbm.at[pl.ds(w * GROUPS_PER_W, GROUPS_PER_W)], src_v)
    pltpu.sync_copy(dst_hbm.at[pl.ds(w * GROUPS_PER_W, GROUPS_PER_W)], dst_v)
    plsc.subcore_barrier()

    for b in range(DEPTH):
        pltpu.async_copy(y_hbm.at[src_v.at[b]], rows_v.at[b], gsems[b])

    lag = NBUF - DEPTH

    @pl.loop(0, GROUPS_PER_W, step=NBUF)
    def _pipe(j):
        for b in range(NBUF):
            g = j + b
            pltpu.make_async_copy(y_hbm.at[src_v.at[g]], rows_v.at[b],
                                  gsems[b]).wait()
            pltpu.async_copy(rows_v.at[b], agg_sh.at[dst_v.at[g]], ssems[b],
                             add=True)
            b2 = (b + DEPTH) % NBUF

            @pl.when(g + DEPTH < GROUPS_PER_W)
            def _prefetch():
                @pl.when(g >= lag)
                def _wait_prev_scatter():
                    pltpu.make_async_copy(rows_v.at[b2],
                                          agg_sh.at[dst_v.at[g - lag]],
                                          ssems[b2]).wait()

                pltpu.async_copy(y_hbm.at[src_v.at[g + DEPTH]], rows_v.at[b2],
                                 gsems[b2])

    for g in range(GROUPS_PER_W - NBUF, GROUPS_PER_W):
        pltpu.make_async_copy(rows_v.at[g % NBUF], agg_sh.at[dst_v.at[g]],
                              ssems[g % NBUF]).wait()

    plsc.subcore_barrier()
    def _out_slice(k):
        return out_hbm.at[pl.ds(w * ROWS_PER_TILE + k * GROUP, GROUP)]

    for k in range(CHUNKS_PER_TILE):
        b = k % NBUF
        if k >= NBUF:
            pltpu.make_async_copy(rows_v.at[b], _out_slice(k - NBUF),
                                  sems[b]).wait()
        pltpu.sync_copy(
            agg_sh.at[pl.ds(s * ROWS_PER_TILE + k * GROUP, GROUP)],
            rows_v.at[b])
        pltpu.async_copy(rows_v.at[b], _out_slice(k), sems[b])
    for k in range(max(0, CHUNKS_PER_TILE - NBUF), CHUNKS_PER_TILE):
        pltpu.make_async_copy(rows_v.at[k % NBUF], _out_slice(k),
                              sems[k % NBUF]).wait()


@functools.cache
def _get_sc_agg():
    return pl.kernel(
        _sc_agg_body,
        out_type=jax.ShapeDtypeStruct((NC * AGG_ROWS, EMB), jnp.float32),
        mesh=plsc.VectorSubcoreMesh(core_axis_name="c", subcore_axis_name="s",
                                    num_cores=NC, num_subcores=NS),
        scratch_types=[
            pltpu.VMEM((GROUPS_PER_W, GROUP), jnp.int32),
            pltpu.VMEM((GROUPS_PER_W, GROUP), jnp.int32),
            pltpu.VMEM((NBUF, GROUP, EMB), jnp.float32),
            pltpu.VMEM_SHARED((AGG_ROWS, EMB), jnp.float32),
        ] + [pltpu.SemaphoreType.DMA] * (2 * NBUF),
        compiler_params=pltpu.CompilerParams(use_tc_tiling_on_sc=False),
    )


def _mm_body(x_ref, w_ref, o_ref):
    o_ref[...] = jnp.dot(x_ref[...], w_ref[...],
                         preferred_element_type=jnp.float32)


N_GROUPS = N_EDGES // GROUP
PAD_GROUPS = E_PAD // GROUP - N_GROUPS


def _pad_idx_body(ei_ref, ps_ref, pd_ref, os_ref, od_ref):
    os_ref[:N_GROUPS] = ei_ref[0]
    os_ref[N_GROUPS:] = ps_ref[...]
    od_ref[:N_GROUPS] = ei_ref[1]
    od_ref[N_GROUPS:] = pd_ref[...]


def _mid_body(p_ref, y_ref, ba_ref, wb_ref, bb_ref, wn_ref, o_ref):
    agg = p_ref[:N_NODES, :] + p_ref[AGG_ROWS:AGG_ROWS + N_NODES, :]
    t = jnp.maximum(y_ref[...] + agg + ba_ref[...], 0.0)
    h = jnp.maximum(
        jnp.dot(t, wb_ref[...], preferred_element_type=jnp.float32)
        + bb_ref[...], 0.0)
    o_ref[...] = jnp.dot(h, wn_ref[...], preferred_element_type=jnp.float32)


def _final_body(p_ref, y_ref, ba_ref, wb_ref, bb_ref, batch_ref,
                wout_ref, bout_ref, o_ref):
    agg = p_ref[:N_NODES, :] + p_ref[AGG_ROWS:AGG_ROWS + N_NODES, :]
    t = jnp.maximum(y_ref[...] + agg + ba_ref[...], 0.0)
    h = jnp.maximum(
        jnp.dot(t, wb_ref[...], preferred_element_type=jnp.float32)
        + bb_ref[...], 0.0)
    gids = lax.broadcasted_iota(jnp.int32, (1, N_GRAPHS), 1)
    onehot = (batch_ref[...] == gids).astype(jnp.float32)
    sums = lax.dot_general(onehot, h, (((0,), (0,)), ((), ())),
                           preferred_element_type=jnp.float32)
    ones = jnp.ones((N_NODES, 1), jnp.float32)
    counts = lax.dot_general(onehot, ones, (((0,), (0,)), ((), ())),
                             preferred_element_type=jnp.float32)
    pooled = sums / jnp.maximum(counts, 1.0)
    o_ref[...] = (jnp.dot(pooled, wout_ref[...],
                          preferred_element_type=jnp.float32) + bout_ref[...])


def kernel(x, edge_index, batch, W1a, b1a, W1b, b1b, W2a, b2a, W2b, b2b,
           Wout, bout):
    pad_n = E_PAD - N_EDGES
    pad_src_c = jnp.asarray(
        (np.arange(pad_n) % N_NODES).astype(np.int32).reshape(-1, GROUP))
    pad_dst_c = jnp.asarray(
        (DUMMY + np.arange(pad_n) % (AGG_ROWS - N_NODES))
        .astype(np.int32).reshape(-1, GROUP))
    i32 = jnp.int32
    src_p, dst_p = pl.pallas_call(
        _pad_idx_body,
        out_shape=[jax.ShapeDtypeStruct((E_PAD // GROUP, GROUP), i32)] * 2)(
            edge_index.reshape(2, N_GROUPS, GROUP), pad_src_c, pad_dst_c)
    zeros = jnp.zeros((GROUP, EMB), jnp.float32)
    b1a2, b1b2, b2a2, b2b2 = (b.reshape(1, EMB) for b in (b1a, b1b, b2a, b2b))
    batch2 = batch.reshape(N_NODES, 1)
    wout_p = jnp.zeros((EMB, 128), jnp.float32).at[:, :N_CLASSES].set(Wout)
    bout_p = jnp.zeros((1, 128), jnp.float32).at[0, :N_CLASSES].set(bout)

    f32 = jnp.float32
    y1 = pl.pallas_call(
        _mm_body,
        out_shape=jax.ShapeDtypeStruct((N_NODES, EMB), f32))(x, W1a)
    p1 = _get_sc_agg()(y1, src_p, dst_p, zeros)
    y2 = pl.pallas_call(
        _mid_body,
        out_shape=jax.ShapeDtypeStruct((N_NODES, EMB), f32))(
            p1, y1, b1a2, W1b, b1b2, W2a)
    p2 = _get_sc_agg()(y2, src_p, dst_p, zeros)
    out_p = pl.pallas_call(
        _final_body,
        out_shape=jax.ShapeDtypeStruct((N_GRAPHS, 128), f32))(
            p2, y2, b2a2, W2b, b2b2, batch2, wout_p, bout_p)
    return out_p[:, :N_CLASSES]
